# baseline (speedup 1.0000x reference)
import functools
import jax
import jax.numpy as jnp
from jax import lax
from jax.experimental import pallas as pl
from jax.experimental.pallas import tpu as pltpu
from jax.experimental.pallas import tpu_sc as plsc

_B, _S, _D = 16, 2048, 512
_NV, _H = 32, 512
_T = _S + _NV
_CH = 80
_NBLK = _B * _T // _CH
_BPB = _T // _CH
_KMAX = _NBLK // 32
_LANES = 16
_NL = _D // _LANES


def _mlp_body(emb_ref, w1_ref, b1_ref, w2_ref, b2_ref, out_ref):
    h = jnp.tanh(
        jnp.dot(emb_ref[...], w1_ref[...],
                preferred_element_type=jnp.float32) + b1_ref[...])
    out_ref[...] = jnp.dot(
        h, w2_ref[...], preferred_element_type=jnp.float32) + b2_ref[...]


def _virtual_rows(embed_table, W1, b1, W2, b2):
    return pl.pallas_call(
        _mlp_body,
        out_shape=jax.ShapeDtypeStruct((_NV, _D), jnp.float32),
    )(embed_table, W1, b1.reshape(1, _H), W2, b2.reshape(1, _D))


def _sc_body(inp_hbm, seq_hbm, virt_hbm, zeros_hbm, out_hbm,
             buf2, vbuf, seqv, zshared, semA, semB, semZ):
    c = lax.axis_index("c")
    s = lax.axis_index("s")
    w = s * 2 + c

    @pl.when(s == 0)
    def _init_zeros():
        pltpu.sync_copy(zeros_hbm, zshared)

    pltpu.sync_copy(seq_hbm, seqv.at[pl.ds(0, 16)])

    infos = []
    for k in range(_KMAX):
        g = w + 32 * k
        bk = g // _BPB
        start = (g % _BPB) * _CH
        lnk = seqv[pl.ds(bk, 16)][0]
        is_copy = (start + _CH) <= lnk
        is_zero = start >= lnk + _NV
        infos.append((bk, start, lnk, is_copy, is_zero))

    any_straddle = jnp.logical_not(infos[0][3] | infos[0][4])
    for k in range(1, _KMAX):
        any_straddle = any_straddle | jnp.logical_not(
            infos[k][3] | infos[k][4])

    @pl.when(any_straddle)
    def _stage_virtual():
        pltpu.sync_copy(virt_hbm, vbuf)

    plsc.subcore_barrier()

    def _in_copy(k, sem):
        bk, start, _, _, _ = infos[k]
        return pltpu.make_async_copy(
            inp_hbm.at[bk, pl.ds(pl.multiple_of(start, 8), _CH), :],
            buf2.at[k % 2], sem)

    @pl.when(infos[0][3])
    def _pro0():
        _in_copy(0, semA).start()

    @pl.when(infos[1][3])
    def _pro1():
        _in_copy(1, semB).start()

    zero16 = jnp.zeros((_LANES,), jnp.float32)
    nzeros = jnp.int32(0)

    for k in range(_KMAX):
        bk, start, lnk, is_copy, is_zero = infos[k]
        sem = semA if k % 2 == 0 else semB
        slot = k % 2

        def _out_dst(bk=bk, start=start):
            return out_hbm.at[bk, pl.ds(pl.multiple_of(start, 8), _CH), :]

        @pl.when(is_copy)
        def _copy(k=k, sem=sem, slot=slot, _out_dst=_out_dst):
            _in_copy(k, sem).wait()
            pltpu.sync_copy(buf2.at[slot], _out_dst())

        @pl.when(is_zero)
        def _zero(_out_dst=_out_dst):
            pltpu.make_async_copy(zshared, _out_dst(), semZ).start()

        nzeros = nzeros + is_zero.astype(jnp.int32)

        @pl.when(jnp.logical_not(is_copy | is_zero))
        def _straddle(bk=bk, start=start, lnk=lnk, slot=slot,
                      _out_dst=_out_dst):
            has_head = lnk > start

            @pl.when(has_head)
            def _head():
                src0 = pl.multiple_of(jnp.minimum(start, _S - _CH), 8)
                delta = start - src0
                pltpu.sync_copy(inp_hbm.at[bk, pl.ds(src0, _CH), :],
                                buf2.at[slot])

                @pl.when(delta > 0)
                def _shift():
                    nshift = jnp.minimum(lnk, _S) - start

                    def _sbody(i, carry):
                        for l in range(_NL):
                            buf2[slot, i, pl.ds(l * _LANES, _LANES)] = (
                                buf2[slot, i + delta,
                                     pl.ds(l * _LANES, _LANES)])
                        return carry
                    lax.fori_loop(0, nshift, _sbody, 0)

            @pl.when(jnp.logical_not(has_head))
            def _zfill():
                pltpu.sync_copy(zeros_hbm, buf2.at[slot])

            v0 = jnp.maximum(lnk, start)
            v1 = jnp.minimum(lnk + _NV, start + _CH)
            pr = v0 - start
            vb0 = v0 - lnk

            def _vbody(i, carry):
                for l in range(_NL):
                    buf2[slot, pr + i, pl.ds(l * _LANES, _LANES)] = (
                        vbuf[vb0 + i, pl.ds(l * _LANES, _LANES)])
                return carry
            lax.fori_loop(0, v1 - v0, _vbody, 0)

            @pl.when(has_head)
            def _ztail():
                zt = jnp.clip(lnk + _NV - start, 0, _CH)

                def _zbody(i, carry):
                    for l in range(_NL):
                        buf2[slot, zt + i, pl.ds(l * _LANES, _LANES)] = (
                            zero16)
                    return carry
                lax.fori_loop(0, _CH - zt, _zbody, 0)

            pltpu.sync_copy(buf2.at[slot], _out_dst())

        if k + 2 < _KMAX:
            @pl.when(infos[k + 2][3])
            def _issue_ahead(k=k, sem=sem):
                _in_copy(k + 2, sem).start()

    def _zdrain(i, carry):
        pltpu.make_async_copy(
            zshared, out_hbm.at[0, pl.ds(0, _CH), :], semZ).wait()
        return carry
    lax.fori_loop(0, nzeros, _zdrain, 0)


@functools.partial(
    pl.kernel,
    out_type=jax.ShapeDtypeStruct((_B, _T, _D), jnp.float32),
    mesh=plsc.VectorSubcoreMesh(core_axis_name="c", subcore_axis_name="s"),
    scratch_types=[
        pltpu.VMEM((2, _CH, _D), jnp.float32),
        pltpu.VMEM((_NV, _D), jnp.float32),
        pltpu.VMEM((48,), jnp.int32),
        pltpu.VMEM_SHARED((_CH, _D), jnp.float32),
        pltpu.SemaphoreType.DMA,
        pltpu.SemaphoreType.DMA,
        pltpu.SemaphoreType.DMA,
    ],
)
def _sc_assemble(inp_hbm, seq_hbm, virt_hbm, zeros_hbm, out_hbm,
                 buf2, vbuf, seqv, zshared, semA, semB, semZ):
    _sc_body(inp_hbm, seq_hbm, virt_hbm, zeros_hbm, out_hbm,
             buf2, vbuf, seqv, zshared, semA, semB, semZ)


def kernel(inputs, seq_len, embed_table, W1, b1, W2, b2):
    seq_len = seq_len.astype(jnp.int32)
    virtual = _virtual_rows(embed_table, W1, b1, W2, b2)
    zeros = jnp.zeros((_CH, _D), jnp.float32)
    out = _sc_assemble(inputs, seq_len, virtual, zeros)
    return out, seq_len + _NV


# speedup vs baseline: 1.1351x; 1.1351x over previous
import functools
import jax
import jax.numpy as jnp
from jax import lax
from jax.experimental import pallas as pl
from jax.experimental.pallas import tpu as pltpu
from jax.experimental.pallas import tpu_sc as plsc

_B, _S, _D = 16, 2048, 512
_NV, _H = 32, 512
_T = _S + _NV
_HALF = _T // 2
_CH = 80
_ZCH = 40
_LANES = 16


def _mlp_body(emb_ref, w1_ref, b1_ref, w2_ref, b2_ref, out_ref):
    h = jnp.tanh(
        jnp.dot(emb_ref[...], w1_ref[...],
                preferred_element_type=jnp.float32) + b1_ref[...])
    out_ref[...] = jnp.dot(
        h, w2_ref[...], preferred_element_type=jnp.float32) + b2_ref[...]


def _virtual_rows(embed_table, W1, b1, W2, b2):
    return pl.pallas_call(
        _mlp_body,
        out_shape=jax.ShapeDtypeStruct((_NV, _D), jnp.float32),
    )(embed_table, W1, b1.reshape(1, _H), W2, b2.reshape(1, _D))


def _sc_body(inp_hbm, seq_hbm, virt_hbm, zeros_hbm, out_hbm,
             buf2, vbuf, zbuf, seqv, semA, semB, semZ, semOutA, semOutB):
    c = lax.axis_index("c")
    s = lax.axis_index("s")
    wid = s * 2 + c
    b = wid % _B
    half = wid // _B
    row0 = half * _HALF
    r1 = row0 + _HALF

    pltpu.sync_copy(seq_hbm, seqv.at[pl.ds(0, 16)])
    pltpu.sync_copy(virt_hbm, vbuf)
    pltpu.sync_copy(zeros_hbm, zbuf)

    ln = seqv[pl.ds(b, 16)][0]

    copy_rows = jnp.clip(ln - row0, 0, _HALF)
    n_full = copy_rows // _CH

    z0 = jnp.clip(ln + _NV, row0, r1)
    zsu = row0 + ((z0 - row0 + _ZCH - 1) // _ZCH) * _ZCH
    nz = (r1 - zsu) // _ZCH

    def _zdst(j):
        zo = pl.multiple_of(zsu + j * _ZCH, 8)
        return out_hbm.at[b, pl.ds(zo, _ZCH), :]

    def _zfire(j, carry):
        pltpu.make_async_copy(zbuf, _zdst(j), semZ).start()
        return carry
    lax.fori_loop(0, nz, _zfire, 0)

    def _src(k):
        ro = pl.multiple_of(row0 + k * _CH, 8)
        return inp_hbm.at[b, pl.ds(ro, _CH), :]

    def _dst(k):
        ro = pl.multiple_of(row0 + k * _CH, 8)
        return out_hbm.at[b, pl.ds(ro, _CH), :]

    @pl.when(n_full > 0)
    def _prologue0():
        pltpu.make_async_copy(_src(0), buf2.at[0], semA).start()

    @pl.when(n_full > 1)
    def _prologue1():
        pltpu.make_async_copy(_src(1), buf2.at[1], semB).start()

    def _cpair(p_, carry):
        k0 = 2 * p_
        k1 = k0 + 1

        @pl.when(k0 > 0)
        def _wout0():
            pltpu.make_async_copy(buf2.at[0], _dst(k0 - 2), semOutA).wait()

        @pl.when(k0 > 0)
        def _start_in0():
            pltpu.make_async_copy(_src(k0), buf2.at[0], semA).start()

        pltpu.make_async_copy(_src(k0), buf2.at[0], semA).wait()
        pltpu.make_async_copy(buf2.at[0], _dst(k0), semOutA).start()

        @pl.when(k1 < n_full)
        def _slot1():
            @pl.when(k1 > 1)
            def _wout1():
                pltpu.make_async_copy(buf2.at[1], _dst(k1 - 2),
                                      semOutB).wait()

            @pl.when(k1 > 1)
            def _start_in1():
                pltpu.make_async_copy(_src(k1), buf2.at[1], semB).start()

            pltpu.make_async_copy(_src(k1), buf2.at[1], semB).wait()
            pltpu.make_async_copy(buf2.at[1], _dst(k1), semOutB).start()
        return carry
    lax.fori_loop(0, (n_full + 1) // 2, _cpair, 0)

    @pl.when(n_full > 0)
    def _drain_out0():
        klast0 = ((n_full - 1) // 2) * 2
        pltpu.make_async_copy(buf2.at[0], _dst(klast0), semOutA).wait()

    @pl.when(n_full > 1)
    def _drain_out1():
        klast1 = ((n_full - 2) // 2) * 2 + 1
        pltpu.make_async_copy(buf2.at[1], _dst(klast1), semOutB).wait()

    owner = jnp.logical_and(ln >= row0, ln < r1)

    @pl.when(owner)
    def _owner_work():
        len8 = (ln // 8) * 8
        pstart = pl.multiple_of(jnp.minimum(len8, _T - _CH), 8)
        p = ln - pstart
        off = row0 + n_full * _CH
        rem8 = pstart - off

        @pl.when(rem8 > 0)
        def _remainder():
            src0 = pl.multiple_of(jnp.minimum(off, _S - _CH), 8)
            delta = off - src0
            pltpu.sync_copy(inp_hbm.at[b, pl.ds(src0, _CH), :], buf2.at[0])
            o = off
            d = delta
            for z in (64, 32, 16, 8):
                take = rem8 & z

                @pl.when(take > 0)
                def _piece(o=o, d=d, z=z):
                    pltpu.sync_copy(
                        buf2.at[0, pl.ds(pl.multiple_of(d, 8), z)],
                        out_hbm.at[b, pl.ds(pl.multiple_of(o, 8), z), :])
                o = o + take
                d = d + take

        pltpu.sync_copy(inp_hbm.at[b, pl.ds(pstart, 48), :],
                        buf2.at[1, pl.ds(0, 48)])

        def _vrow(j, carry):
            for l in range(_D // _LANES):
                buf2[1, p + j, pl.ds(l * _LANES, _LANES)] = (
                    vbuf[j, pl.ds(l * _LANES, _LANES)])
            return carry
        lax.fori_loop(0, _NV, _vrow, 0)

        zero16 = jnp.zeros((_LANES,), jnp.float32)

        def _zrow(j, carry):
            for l in range(_D // _LANES):
                buf2[1, p + _NV + j, pl.ds(l * _LANES, _LANES)] = zero16
            return carry
        lax.fori_loop(0, _CH - _NV - p, _zrow, 0)

        pltpu.sync_copy(buf2.at[1], out_hbm.at[b, pl.ds(pstart, _CH), :])

    def _zdrain(j, carry):
        pltpu.make_async_copy(zbuf, _zdst(j), semZ).wait()
        return carry
    lax.fori_loop(0, nz, _zdrain, 0)


@functools.partial(
    pl.kernel,
    out_type=jax.ShapeDtypeStruct((_B, _T, _D), jnp.float32),
    mesh=plsc.VectorSubcoreMesh(core_axis_name="c", subcore_axis_name="s"),
    scratch_types=[
        pltpu.VMEM((2, _CH, _D), jnp.float32),
        pltpu.VMEM((_NV, _D), jnp.float32),
        pltpu.VMEM((_ZCH, _D), jnp.float32),
        pltpu.VMEM((48,), jnp.int32),
        pltpu.SemaphoreType.DMA,
        pltpu.SemaphoreType.DMA,
        pltpu.SemaphoreType.DMA,
        pltpu.SemaphoreType.DMA,
        pltpu.SemaphoreType.DMA,
    ],
)
def _sc_assemble(inp_hbm, seq_hbm, virt_hbm, zeros_hbm, out_hbm,
                 buf2, vbuf, zbuf, seqv, semA, semB, semZ, semOutA, semOutB):
    _sc_body(inp_hbm, seq_hbm, virt_hbm, zeros_hbm, out_hbm,
             buf2, vbuf, zbuf, seqv, semA, semB, semZ, semOutA, semOutB)


def kernel(inputs, seq_len, embed_table, W1, b1, W2, b2):
    seq_len = seq_len.astype(jnp.int32)
    virtual = _virtual_rows(embed_table, W1, b1, W2, b2)
    zeros = jnp.zeros((_ZCH, _D), jnp.float32)
    out = _sc_assemble(inputs, seq_len, virtual, zeros)
    return out, seq_len + _NV


# speedup vs baseline: 1.1610x; 1.0228x over previous
import functools
import jax
import jax.numpy as jnp
from jax import lax
from jax.experimental import pallas as pl
from jax.experimental.pallas import tpu as pltpu
from jax.experimental.pallas import tpu_sc as plsc

_B, _S, _D = 16, 2048, 512
_NV, _H = 32, 512
_T = _S + _NV
_HALF = _T // 2
_CH = 80
_ZCH = 40
_LANES = 16


def _mlp_body(emb_ref, w1_ref, b1_ref, w2_ref, b2_ref, out_ref):
    h = jnp.tanh(
        jnp.dot(emb_ref[...], w1_ref[...],
                preferred_element_type=jnp.float32) + b1_ref[...])
    out_ref[...] = jnp.dot(
        h, w2_ref[...], preferred_element_type=jnp.float32) + b2_ref[...]


def _virtual_rows(embed_table, W1, b1, W2, b2):
    return pl.pallas_call(
        _mlp_body,
        out_shape=jax.ShapeDtypeStruct((_NV, _D), jnp.float32),
    )(embed_table, W1, b1.reshape(1, _H), W2, b2.reshape(1, _D))


def _sc_body(inp_hbm, seq_hbm, virt_hbm, zeros_hbm, out_hbm,
             buf2, vbuf, zbuf, seqv, semA, semB, semZ, semOutA, semOutB):
    c = lax.axis_index("c")
    s = lax.axis_index("s")
    wid = s * 2 + c
    b = wid % _B
    half = wid // _B
    row0 = half * _HALF
    r1 = row0 + _HALF

    @pl.when(s == 0)
    def _init_zshared():
        pltpu.sync_copy(zeros_hbm, zbuf)

    pltpu.sync_copy(seq_hbm, seqv.at[pl.ds(0, 16)])
    pltpu.sync_copy(virt_hbm, vbuf)
    plsc.subcore_barrier()

    ln = seqv[pl.ds(b, 16)][0]

    copy_rows = jnp.clip(ln - row0, 0, _HALF)
    n_full = copy_rows // _CH

    z0 = jnp.clip(ln + _NV, row0, r1)
    zsu = row0 + ((z0 - row0 + _ZCH - 1) // _ZCH) * _ZCH
    nz = (r1 - zsu) // _ZCH

    def _zdst(j):
        zo = pl.multiple_of(zsu + j * _ZCH, 8)
        return out_hbm.at[b, pl.ds(zo, _ZCH), :]

    def _zfire(j, carry):
        pltpu.make_async_copy(zbuf, _zdst(j), semZ).start()
        return carry
    lax.fori_loop(0, nz, _zfire, 0)

    def _src(k):
        ro = pl.multiple_of(row0 + k * _CH, 8)
        return inp_hbm.at[b, pl.ds(ro, _CH), :]

    def _dst(k):
        ro = pl.multiple_of(row0 + k * _CH, 8)
        return out_hbm.at[b, pl.ds(ro, _CH), :]

    @pl.when(n_full > 0)
    def _prologue0():
        pltpu.make_async_copy(_src(0), buf2.at[0], semA).start()

    @pl.when(n_full > 1)
    def _prologue1():
        pltpu.make_async_copy(_src(1), buf2.at[1], semB).start()

    def _cpair(p_, carry):
        k0 = 2 * p_
        k1 = k0 + 1

        @pl.when(k0 > 0)
        def _wout0():
            pltpu.make_async_copy(buf2.at[0], _dst(k0 - 2), semOutA).wait()

        @pl.when(k0 > 0)
        def _start_in0():
            pltpu.make_async_copy(_src(k0), buf2.at[0], semA).start()

        pltpu.make_async_copy(_src(k0), buf2.at[0], semA).wait()
        pltpu.make_async_copy(buf2.at[0], _dst(k0), semOutA).start()

        @pl.when(k1 < n_full)
        def _slot1():
            @pl.when(k1 > 1)
            def _wout1():
                pltpu.make_async_copy(buf2.at[1], _dst(k1 - 2),
                                      semOutB).wait()

            @pl.when(k1 > 1)
            def _start_in1():
                pltpu.make_async_copy(_src(k1), buf2.at[1], semB).start()

            pltpu.make_async_copy(_src(k1), buf2.at[1], semB).wait()
            pltpu.make_async_copy(buf2.at[1], _dst(k1), semOutB).start()
        return carry
    lax.fori_loop(0, (n_full + 1) // 2, _cpair, 0)

    @pl.when(n_full > 0)
    def _drain_out0():
        klast0 = ((n_full - 1) // 2) * 2
        pltpu.make_async_copy(buf2.at[0], _dst(klast0), semOutA).wait()

    @pl.when(n_full > 1)
    def _drain_out1():
        klast1 = ((n_full - 2) // 2) * 2 + 1
        pltpu.make_async_copy(buf2.at[1], _dst(klast1), semOutB).wait()

    owner = jnp.logical_and(ln >= row0, ln < r1)

    @pl.when(owner)
    def _owner_work():
        len8 = (ln // 8) * 8
        pstart = pl.multiple_of(jnp.minimum(len8, _T - _CH), 8)
        p = ln - pstart
        off = row0 + n_full * _CH
        rem8 = pstart - off

        @pl.when(rem8 > 0)
        def _remainder():
            src0 = pl.multiple_of(jnp.minimum(off, _S - _CH), 8)
            delta = off - src0
            pltpu.sync_copy(inp_hbm.at[b, pl.ds(src0, _CH), :], buf2.at[0])
            o = off
            d = delta
            for z in (64, 32, 16, 8):
                take = rem8 & z

                @pl.when(take > 0)
                def _piece(o=o, d=d, z=z):
                    pltpu.sync_copy(
                        buf2.at[0, pl.ds(pl.multiple_of(d, 8), z)],
                        out_hbm.at[b, pl.ds(pl.multiple_of(o, 8), z), :])
                o = o + take
                d = d + take

        pltpu.sync_copy(inp_hbm.at[b, pl.ds(pstart, 48), :],
                        buf2.at[1, pl.ds(0, 48)])

        def _vrow(j, carry):
            for l in range(_D // _LANES):
                buf2[1, p + j, pl.ds(l * _LANES, _LANES)] = (
                    vbuf[j, pl.ds(l * _LANES, _LANES)])
            return carry
        lax.fori_loop(0, _NV, _vrow, 0)

        zero16 = jnp.zeros((_LANES,), jnp.float32)

        def _zrow(j, carry):
            for l in range(_D // _LANES):
                buf2[1, p + _NV + j, pl.ds(l * _LANES, _LANES)] = zero16
            return carry
        lax.fori_loop(0, _CH - _NV - p, _zrow, 0)

        pltpu.sync_copy(buf2.at[1], out_hbm.at[b, pl.ds(pstart, _CH), :])

    def _zdrain(j, carry):
        pltpu.make_async_copy(zbuf, _zdst(j), semZ).wait()
        return carry
    lax.fori_loop(0, nz, _zdrain, 0)


@functools.partial(
    pl.kernel,
    out_type=jax.ShapeDtypeStruct((_B, _T, _D), jnp.float32),
    mesh=plsc.VectorSubcoreMesh(core_axis_name="c", subcore_axis_name="s"),
    scratch_types=[
        pltpu.VMEM((2, _CH, _D), jnp.float32),
        pltpu.VMEM((_NV, _D), jnp.float32),
        pltpu.VMEM_SHARED((_ZCH, _D), jnp.float32),
        pltpu.VMEM((48,), jnp.int32),
        pltpu.SemaphoreType.DMA,
        pltpu.SemaphoreType.DMA,
        pltpu.SemaphoreType.DMA,
        pltpu.SemaphoreType.DMA,
        pltpu.SemaphoreType.DMA,
    ],
)
def _sc_assemble(inp_hbm, seq_hbm, virt_hbm, zeros_hbm, out_hbm,
                 buf2, vbuf, zbuf, seqv, semA, semB, semZ, semOutA, semOutB):
    _sc_body(inp_hbm, seq_hbm, virt_hbm, zeros_hbm, out_hbm,
             buf2, vbuf, zbuf, seqv, semA, semB, semZ, semOutA, semOutB)


def kernel(inputs, seq_len, embed_table, W1, b1, W2, b2):
    seq_len = seq_len.astype(jnp.int32)
    virtual = _virtual_rows(embed_table, W1, b1, W2, b2)
    zeros = jnp.zeros((_ZCH, _D), jnp.float32)
    out = _sc_assemble(inputs, seq_len, virtual, zeros)
    return out, seq_len + _NV
